# Initial kernel scaffold; baseline (speedup 1.0000x reference)
import jax
import jax.numpy as jnp
from jax.experimental import pallas as pl


def kernel(query, reference_points, input_flatten, input_spatial_shapes, input_level_start_index, q_lidar_indices, W_value, b_value, W_off, b_off, W_attn, b_attn, W_out, b_out):
    raise NotImplementedError("write your pallas kernel here")


# speedup vs baseline: 26.7611x; 26.7611x over previous
import functools

import jax
import jax.numpy as jnp
import numpy as np
from jax import lax
from jax.experimental import pallas as pl
from jax.experimental.pallas import tpu as pltpu
from jax.experimental.pallas import tpu_sc as plsc

_DM = 256
_M = 8
_L = 4
_P = 4
_Dh = 32
_LP = _L * _P


def _mm_bias(x, w, b, blk_rows):
    R, K = x.shape
    C = w.shape[1]

    def body(x_ref, w_ref, b_ref, o_ref):
        o_ref[...] = (
            jnp.dot(x_ref[...], w_ref[...], preferred_element_type=jnp.float32)
            + b_ref[...]
        )

    return pl.pallas_call(
        body,
        grid=(R // blk_rows,),
        in_specs=[
            pl.BlockSpec((blk_rows, K), lambda i: (i, 0)),
            pl.BlockSpec((K, C), lambda i: (0, 0)),
            pl.BlockSpec((1, C), lambda i: (0, 0)),
        ],
        out_specs=pl.BlockSpec((blk_rows, C), lambda i: (i, 0)),
        out_shape=jax.ShapeDtypeStruct((R, C), jnp.float32),
    )(x, w, b.reshape(1, C))


def _softmax16(x):
    NQ = x.shape[0]
    R = NQ * _M
    x = x.reshape(R, _LP)

    def body(x_ref, o_ref):
        v = x_ref[...]
        v = v - jnp.max(v, axis=-1, keepdims=True)
        e = jnp.exp(v)
        o_ref[...] = e / jnp.sum(e, axis=-1, keepdims=True)

    out = pl.pallas_call(
        body,
        grid=(R // 2048,),
        in_specs=[pl.BlockSpec((2048, _LP), lambda i: (i, 0))],
        out_specs=pl.BlockSpec((2048, _LP), lambda i: (i, 0)),
        out_shape=jax.ShapeDtypeStruct((R, _LP), jnp.float32),
    )(x)
    return out.reshape(NQ, _M * _LP)


def _sc_attn(off, probs, refx, ci, table, NJ, LEN, Lq):
    NW = 32
    JPW = NJ // NW
    QPW = JPW // _M
    mesh = plsc.VectorSubcoreMesh(core_axis_name="c", subcore_axis_name="s",
                                  num_cores=2, num_subcores=16)

    @functools.partial(
        pl.kernel,
        out_type=jax.ShapeDtypeStruct((NJ, _Dh), jnp.float32),
        mesh=mesh,
        compiler_params=pltpu.CompilerParams(use_tc_tiling_on_sc=False),
        scratch_types=[
            pltpu.VMEM((QPW, 256), jnp.float32),
            pltpu.VMEM((QPW, 128), jnp.float32),
            pltpu.VMEM((QPW, 32), jnp.float32),
            pltpu.VMEM((3, 16), jnp.int32),
            pltpu.VMEM((64,), jnp.int32),
            pltpu.VMEM((64,), jnp.float32),
            pltpu.VMEM((64, _Dh), jnp.float32),
            pltpu.VMEM((JPW, _Dh), jnp.float32),
            pltpu.SemaphoreType.DMA,
        ],
    )
    def k(off_hbm, aw_hbm, ref_hbm, ci_hbm, table_hbm, out_hbm,
          offv, awv, refv, civ, idxv, wv, rowsv, outv, sem):
        c = lax.axis_index("c")
        s = lax.axis_index("s")
        wid = s * 2 + c
        qrow = wid * QPW
        n = qrow // Lq
        pltpu.sync_copy(off_hbm.at[pl.ds(qrow, QPW)], offv)
        pltpu.sync_copy(aw_hbm.at[pl.ds(qrow, QPW)], awv)
        pltpu.sync_copy(ref_hbm.at[pl.ds(qrow, QPW)], refv)
        pltpu.sync_copy(ci_hbm, civ)
        Wi = civ[0, :]
        Hi = civ[1, :]
        Si = civ[2, :]
        Wf = Wi.astype(jnp.float32)
        Hf = Hi.astype(jnp.float32)
        Wm1 = Wi - 1
        Hm1 = Hi - 1
        nm0 = n * (LEN * _M)

        def body(jj, carry):
            qo = jj // _M
            m = jj - qo * _M
            offx = offv[qo, pl.ds(m * 32, 16)]
            offy = offv[qo, pl.ds(m * 32 + 16, 16)]
            aw = awv[qo, pl.ds(m * 16, 16)]
            rx = refv[qo, pl.ds(0, 16)]
            ry = refv[qo, pl.ds(16, 16)]
            x = rx * Wf + offx - 0.5
            y = ry * Hf + offy - 0.5
            xi = (x + 512.0).astype(jnp.int32)
            yi = (y + 512.0).astype(jnp.int32)
            x0i = xi - 512
            y0i = yi - 512
            fx = x - x0i.astype(jnp.float32)
            fy = y - y0i.astype(jnp.float32)
            gx0 = 1.0 - fx
            gy0 = 1.0 - fy
            for t2, (dy, dx) in enumerate(((0, 0), (0, 1), (1, 0), (1, 1))):
                ixi = x0i + dx
                iyi = y0i + dy
                ok = (ixi >= 0) & (ixi <= Wm1) & (iyi >= 0) & (iyi <= Hm1)
                ixc = jnp.minimum(jnp.maximum(ixi, 0), Wm1)
                iyc = jnp.minimum(jnp.maximum(iyi, 0), Hm1)
                row = Si + iyc * Wi + ixc
                g = (nm0 + m) + row * _M
                wt = ((fx if dx else gx0) * (fy if dy else gy0)
                      * aw * jnp.where(ok, 1.0, 0.0))
                idxv[pl.ds(t2 * 16, 16)] = g
                wv[pl.ds(t2 * 16, 16)] = wt
            pltpu.async_copy(table_hbm.at[idxv], rowsv, sem).wait()

            z = jnp.zeros((16,), jnp.float32)
            a0, a1 = z, z
            for t2 in range(4):
                wvec = wv[pl.ds(t2 * 16, 16)]
                for sl in range(16):
                    t = t2 * 16 + sl
                    wsc = wvec[sl]
                    a0 = a0 + wsc * rowsv[t, pl.ds(0, 16)]
                    a1 = a1 + wsc * rowsv[t, pl.ds(16, 16)]
            outv[jj, pl.ds(0, 16)] = a0
            outv[jj, pl.ds(16, 16)] = a1
            return carry

        lax.fori_loop(0, JPW, body, jnp.int32(0))
        pltpu.sync_copy(outv, out_hbm.at[pl.ds(wid * JPW, JPW)])

    return k(off, probs, refx, ci, table)


def kernel(query, reference_points, input_flatten, input_spatial_shapes,
           input_level_start_index, q_lidar_indices, W_value, b_value,
           W_off, b_off, W_attn, b_attn, W_out, b_out):
    N, Lq, _ = query.shape
    LEN = input_flatten.shape[1]
    NJ = N * Lq * _M

    value = _mm_bias(input_flatten.reshape(N * LEN, _DM), W_value.T, b_value,
                     (N * LEN) // 17)
    table = value.reshape(N * LEN * _M, _Dh)

    Wofp = (W_off.reshape(_M, _L, _P, 2, _DM)
            .transpose(0, 3, 1, 2, 4).reshape(_M * _LP * 2, _DM))
    bofp = b_off.reshape(_M, _L, _P, 2).transpose(0, 3, 1, 2).reshape(-1)
    Wcat = jnp.concatenate([Wofp, W_attn], axis=0).T
    bcat = jnp.concatenate([bofp, b_attn], axis=0)
    ol = _mm_bias(query.reshape(N * Lq, _DM), Wcat, bcat, Lq)
    off = ol[:, :_M * _LP * 2]
    probs = _softmax16(ol[:, _M * _LP * 2:])

    refx = jnp.repeat(reference_points.transpose(0, 1, 3, 2), _P,
                      axis=-1).reshape(N * Lq, 2 * _LP)
    ss = input_spatial_shapes.astype(jnp.int32)
    lsi = input_level_start_index.astype(jnp.int32)
    ci = jnp.stack([
        jnp.repeat(ss[:, 1], _P),
        jnp.repeat(ss[:, 0], _P),
        jnp.repeat(lsi, _P),
    ])

    attn_out = _sc_attn(off, probs, refx, ci, table, NJ, LEN, Lq)
    out_i = attn_out.reshape(N * Lq, _M * _Dh)
    return _mm_bias(out_i, W_out.T, b_out, Lq).reshape(N, Lq, _DM)


# speedup vs baseline: 46.2827x; 1.7295x over previous
import functools

import jax
import jax.numpy as jnp
import numpy as np
from jax import lax
from jax.experimental import pallas as pl
from jax.experimental.pallas import tpu as pltpu
from jax.experimental.pallas import tpu_sc as plsc

_DM = 256
_M = 8
_L = 4
_P = 4
_Dh = 32
_LP = _L * _P


def _mm_bias(x, w, b, blk_rows):
    R, K = x.shape
    C = w.shape[1]

    def body(x_ref, w_ref, b_ref, o_ref):
        o_ref[...] = (
            jnp.dot(x_ref[...], w_ref[...], preferred_element_type=jnp.float32)
            + b_ref[...]
        )

    return pl.pallas_call(
        body,
        grid=(R // blk_rows,),
        in_specs=[
            pl.BlockSpec((blk_rows, K), lambda i: (i, 0)),
            pl.BlockSpec((K, C), lambda i: (0, 0)),
            pl.BlockSpec((1, C), lambda i: (0, 0)),
        ],
        out_specs=pl.BlockSpec((blk_rows, C), lambda i: (i, 0)),
        out_shape=jax.ShapeDtypeStruct((R, C), jnp.float32),
    )(x, w, b.reshape(1, C))


def _softmax16(x):
    NQ = x.shape[0]
    R = NQ * _M
    x = x.reshape(R, _LP)

    def body(x_ref, o_ref):
        v = x_ref[...]
        v = v - jnp.max(v, axis=-1, keepdims=True)
        e = jnp.exp(v)
        o_ref[...] = e / jnp.sum(e, axis=-1, keepdims=True)

    out = pl.pallas_call(
        body,
        grid=(R // 2048,),
        in_specs=[pl.BlockSpec((2048, _LP), lambda i: (i, 0))],
        out_specs=pl.BlockSpec((2048, _LP), lambda i: (i, 0)),
        out_shape=jax.ShapeDtypeStruct((R, _LP), jnp.float32),
    )(x)
    return out.reshape(NQ, _M * _LP)


def _sc_attn(off, probs, refx, ci, table, NJ, LEN, Lq):
    NW = 32
    JPW = NJ // NW
    QPW = JPW // _M
    mesh = plsc.VectorSubcoreMesh(core_axis_name="c", subcore_axis_name="s",
                                  num_cores=2, num_subcores=16)

    @functools.partial(
        pl.kernel,
        out_type=jax.ShapeDtypeStruct((NJ, _Dh), jnp.float32),
        mesh=mesh,
        compiler_params=pltpu.CompilerParams(use_tc_tiling_on_sc=False),
        scratch_types=[
            pltpu.VMEM((QPW, 256), jnp.float32),
            pltpu.VMEM((QPW, 128), jnp.float32),
            pltpu.VMEM((QPW, 32), jnp.float32),
            pltpu.VMEM((3, 16), jnp.int32),
            pltpu.VMEM((64,), jnp.int32),
            pltpu.VMEM((64,), jnp.int32),
            pltpu.VMEM((64, _Dh), jnp.float32),
            pltpu.VMEM((64, _Dh), jnp.float32),
            pltpu.VMEM((JPW, _Dh), jnp.float32),
            pltpu.SemaphoreType.DMA,
            pltpu.SemaphoreType.DMA,
        ],
    )
    def k(off_hbm, aw_hbm, ref_hbm, ci_hbm, table_hbm, out_hbm,
          offv, awv, refv, civ, idx0, idx1, rows0, rows1, outv, sem0, sem1):
        c = lax.axis_index("c")
        s = lax.axis_index("s")
        wid = s * 2 + c
        qrow = wid * QPW
        n = qrow // Lq
        pltpu.sync_copy(off_hbm.at[pl.ds(qrow, QPW)], offv)
        pltpu.sync_copy(aw_hbm.at[pl.ds(qrow, QPW)], awv)
        pltpu.sync_copy(ref_hbm.at[pl.ds(qrow, QPW)], refv)
        pltpu.sync_copy(ci_hbm, civ)
        Wi = civ[0, :]
        Hi = civ[1, :]
        Si = civ[2, :]
        Wf = Wi.astype(jnp.float32)
        Hf = Hi.astype(jnp.float32)
        Wm1 = Wi - 1
        Hm1 = Hi - 1
        nm0 = n * (LEN * _M)

        def compute_issue(jj, idxb, rowsb, semb):
            qo = jj // _M
            m = jj - qo * _M
            offx = offv[qo, pl.ds(m * 32, 16)]
            offy = offv[qo, pl.ds(m * 32 + 16, 16)]
            aw = awv[qo, pl.ds(m * 16, 16)]
            rx = refv[qo, pl.ds(0, 16)]
            ry = refv[qo, pl.ds(16, 16)]
            x = rx * Wf + offx - 0.5
            y = ry * Hf + offy - 0.5
            xi = (x + 512.0).astype(jnp.int32)
            yi = (y + 512.0).astype(jnp.int32)
            x0i = xi - 512
            y0i = yi - 512
            fx = x - x0i.astype(jnp.float32)
            fy = y - y0i.astype(jnp.float32)
            gx0 = 1.0 - fx
            gy0 = 1.0 - fy
            ws = []
            for t2, (dy, dx) in enumerate(((0, 0), (0, 1), (1, 0), (1, 1))):
                ixi = x0i + dx
                iyi = y0i + dy
                ok = (ixi >= 0) & (ixi <= Wm1) & (iyi >= 0) & (iyi <= Hm1)
                ixc = jnp.minimum(jnp.maximum(ixi, 0), Wm1)
                iyc = jnp.minimum(jnp.maximum(iyi, 0), Hm1)
                row = Si + iyc * Wi + ixc
                g = (nm0 + m) + row * _M
                wt = ((fx if dx else gx0) * (fy if dy else gy0)
                      * aw * jnp.where(ok, 1.0, 0.0))
                idxb[pl.ds(t2 * 16, 16)] = g
                ws.append(wt)
            cp = pltpu.async_copy(table_hbm.at[idxb], rowsb, semb)
            del cp
            return tuple(ws)

        def drain(jj, ws, idxb, rowsb, semb):
            pltpu.make_async_copy(table_hbm.at[idxb], rowsb, semb).wait()
            z = jnp.zeros((16,), jnp.float32)
            a0, a1 = z, z
            for t2 in range(4):
                wvec = ws[t2]
                for sl in range(16):
                    t = t2 * 16 + sl
                    wsc = wvec[sl]
                    a0 = a0 + wsc * rowsb[t, pl.ds(0, 16)]
                    a1 = a1 + wsc * rowsb[t, pl.ds(16, 16)]
            outv[jj, pl.ds(0, 16)] = a0
            outv[jj, pl.ds(16, 16)] = a1

        w_first = compute_issue(0, idx0, rows0, sem0)

        def body(jp, w0):
            j0 = 2 * jp
            w1 = compute_issue(j0 + 1, idx1, rows1, sem1)
            drain(j0, w0, idx0, rows0, sem0)
            w2 = compute_issue(j0 + 2, idx0, rows0, sem0)
            drain(j0 + 1, w1, idx1, rows1, sem1)
            return w2

        w_last = lax.fori_loop(0, JPW // 2 - 1, body, w_first)
        jl = JPW - 2
        w1 = compute_issue(jl + 1, idx1, rows1, sem1)
        drain(jl, w_last, idx0, rows0, sem0)
        drain(jl + 1, w1, idx1, rows1, sem1)
        pltpu.sync_copy(outv, out_hbm.at[pl.ds(wid * JPW, JPW)])

    return k(off, probs, refx, ci, table)


def kernel(query, reference_points, input_flatten, input_spatial_shapes,
           input_level_start_index, q_lidar_indices, W_value, b_value,
           W_off, b_off, W_attn, b_attn, W_out, b_out):
    N, Lq, _ = query.shape
    LEN = input_flatten.shape[1]
    NJ = N * Lq * _M

    value = _mm_bias(input_flatten.reshape(N * LEN, _DM), W_value.T, b_value,
                     (N * LEN) // 17)
    table = value.reshape(N * LEN * _M, _Dh)

    Wofp = (W_off.reshape(_M, _L, _P, 2, _DM)
            .transpose(0, 3, 1, 2, 4).reshape(_M * _LP * 2, _DM))
    bofp = b_off.reshape(_M, _L, _P, 2).transpose(0, 3, 1, 2).reshape(-1)
    Wcat = jnp.concatenate([Wofp, W_attn], axis=0).T
    bcat = jnp.concatenate([bofp, b_attn], axis=0)
    ol = _mm_bias(query.reshape(N * Lq, _DM), Wcat, bcat, Lq)
    off = ol[:, :_M * _LP * 2]
    probs = _softmax16(ol[:, _M * _LP * 2:])

    refx = jnp.repeat(reference_points.transpose(0, 1, 3, 2), _P,
                      axis=-1).reshape(N * Lq, 2 * _LP)
    ss = input_spatial_shapes.astype(jnp.int32)
    lsi = input_level_start_index.astype(jnp.int32)
    ci = jnp.stack([
        jnp.repeat(ss[:, 1], _P),
        jnp.repeat(ss[:, 0], _P),
        jnp.repeat(lsi, _P),
    ])

    attn_out = _sc_attn(off, probs, refx, ci, table, NJ, LEN, Lq)
    out_i = attn_out.reshape(N * Lq, _M * _Dh)
    return _mm_bias(out_i, W_out.T, b_out, Lq).reshape(N, Lq, _DM)


# speedup vs baseline: 49.9561x; 1.0794x over previous
import functools

import jax
import jax.numpy as jnp
import numpy as np
from jax import lax
from jax.experimental import pallas as pl
from jax.experimental.pallas import tpu as pltpu
from jax.experimental.pallas import tpu_sc as plsc

_DM = 256
_M = 8
_L = 4
_P = 4
_Dh = 32
_LP = _L * _P


def _mm_bias(x, w, b, blk_rows):
    R, K = x.shape
    C = w.shape[1]

    def body(x_ref, w_ref, b_ref, o_ref):
        o_ref[...] = (
            jnp.dot(x_ref[...], w_ref[...], preferred_element_type=jnp.float32)
            + b_ref[...]
        )

    return pl.pallas_call(
        body,
        grid=(R // blk_rows,),
        in_specs=[
            pl.BlockSpec((blk_rows, K), lambda i: (i, 0)),
            pl.BlockSpec((K, C), lambda i: (0, 0)),
            pl.BlockSpec((1, C), lambda i: (0, 0)),
        ],
        out_specs=pl.BlockSpec((blk_rows, C), lambda i: (i, 0)),
        out_shape=jax.ShapeDtypeStruct((R, C), jnp.float32),
    )(x, w, b.reshape(1, C))


def _qprep(q2, rp8, Wcat, bcat, E, Smat):
    R = q2.shape[0]
    C = Wcat.shape[1]
    BO = C - _M * _LP

    def body(q_ref, rp_ref, w_ref, b_ref, e_ref, s_ref, o_ref, p_ref, r_ref):
        t = (jnp.dot(q_ref[...], w_ref[...],
                     preferred_element_type=jnp.float32) + b_ref[...])
        lg = t[:, BO:]
        ex = jnp.exp(lg - jnp.max(lg, axis=-1, keepdims=True))
        den = jnp.dot(ex, s_ref[...], preferred_element_type=jnp.float32,
                      precision=jax.lax.Precision.HIGHEST)
        o_ref[...] = t[:, :BO]
        p_ref[...] = ex / den
        r_ref[...] = jnp.dot(rp_ref[...], e_ref[...],
                             preferred_element_type=jnp.float32,
                             precision=jax.lax.Precision.HIGHEST)

    blk = 1024
    return pl.pallas_call(
        body,
        grid=(R // blk,),
        in_specs=[
            pl.BlockSpec((blk, q2.shape[1]), lambda i: (i, 0)),
            pl.BlockSpec((blk, 128), lambda i: (i, 0)),
            pl.BlockSpec(Wcat.shape, lambda i: (0, 0)),
            pl.BlockSpec((1, C), lambda i: (0, 0)),
            pl.BlockSpec(E.shape, lambda i: (0, 0)),
            pl.BlockSpec(Smat.shape, lambda i: (0, 0)),
        ],
        out_specs=[
            pl.BlockSpec((blk, BO), lambda i: (i, 0)),
            pl.BlockSpec((blk, C - BO), lambda i: (i, 0)),
            pl.BlockSpec((blk, 2 * _LP), lambda i: (i, 0)),
        ],
        out_shape=[
            jax.ShapeDtypeStruct((R, BO), jnp.float32),
            jax.ShapeDtypeStruct((R, C - BO), jnp.float32),
            jax.ShapeDtypeStruct((R, 2 * _LP), jnp.float32),
        ],
    )(q2, rp8, Wcat, bcat.reshape(1, C), E, Smat)


def _sc_attn(off, probs, refx, ci, table, NJ, LEN, Lq):
    NW = 32
    JPW = NJ // NW
    QPW = JPW // _M
    mesh = plsc.VectorSubcoreMesh(core_axis_name="c", subcore_axis_name="s",
                                  num_cores=2, num_subcores=16)

    @functools.partial(
        pl.kernel,
        out_type=jax.ShapeDtypeStruct((NJ, _Dh), jnp.float32),
        mesh=mesh,
        compiler_params=pltpu.CompilerParams(use_tc_tiling_on_sc=False),
        scratch_types=[
            pltpu.VMEM((QPW, 256), jnp.float32),
            pltpu.VMEM((QPW, 128), jnp.float32),
            pltpu.VMEM((QPW, 32), jnp.float32),
            pltpu.VMEM((3, 16), jnp.int32),
            pltpu.VMEM((64,), jnp.int32),
            pltpu.VMEM((64,), jnp.int32),
            pltpu.VMEM((64,), jnp.int32),
            pltpu.VMEM((64,), jnp.int32),
            pltpu.VMEM((64, _Dh), jnp.float32),
            pltpu.VMEM((64, _Dh), jnp.float32),
            pltpu.VMEM((64, _Dh), jnp.float32),
            pltpu.VMEM((64, _Dh), jnp.float32),
            pltpu.VMEM((JPW, _Dh), jnp.float32),
            pltpu.SemaphoreType.DMA,
            pltpu.SemaphoreType.DMA,
            pltpu.SemaphoreType.DMA,
            pltpu.SemaphoreType.DMA,
        ],
    )
    def k(off_hbm, aw_hbm, ref_hbm, ci_hbm, table_hbm, out_hbm,
          offv, awv, refv, civ, idx0, idx1, idx2, idx3,
          rows0, rows1, rows2, rows3, outv, sem0, sem1, sem2, sem3):
        c = lax.axis_index("c")
        s = lax.axis_index("s")
        wid = s * 2 + c
        qrow = wid * QPW
        n = qrow // Lq
        pltpu.sync_copy(off_hbm.at[pl.ds(qrow, QPW)], offv)
        pltpu.sync_copy(aw_hbm.at[pl.ds(qrow, QPW)], awv)
        pltpu.sync_copy(ref_hbm.at[pl.ds(qrow, QPW)], refv)
        pltpu.sync_copy(ci_hbm, civ)
        Wi = civ[0, :]
        Hi = civ[1, :]
        Si = civ[2, :]
        Wf = Wi.astype(jnp.float32)
        Hf = Hi.astype(jnp.float32)
        Wm1 = Wi - 1
        Hm1 = Hi - 1
        nm0 = n * (LEN * _M)

        def compute_issue(jj, idxb, rowsb, semb):
            qo = jj // _M
            m = jj - qo * _M
            offx = offv[qo, pl.ds(m * 32, 16)]
            offy = offv[qo, pl.ds(m * 32 + 16, 16)]
            aw = awv[qo, pl.ds(m * 16, 16)]
            rx = refv[qo, pl.ds(0, 16)]
            ry = refv[qo, pl.ds(16, 16)]
            x = rx * Wf + offx - 0.5
            y = ry * Hf + offy - 0.5
            xi = (x + 512.0).astype(jnp.int32)
            yi = (y + 512.0).astype(jnp.int32)
            x0i = xi - 512
            y0i = yi - 512
            fx = x - x0i.astype(jnp.float32)
            fy = y - y0i.astype(jnp.float32)
            gx0 = 1.0 - fx
            gy0 = 1.0 - fy
            ws = []
            for t2, (dy, dx) in enumerate(((0, 0), (0, 1), (1, 0), (1, 1))):
                ixi = x0i + dx
                iyi = y0i + dy
                ok = (ixi >= 0) & (ixi <= Wm1) & (iyi >= 0) & (iyi <= Hm1)
                ixc = jnp.minimum(jnp.maximum(ixi, 0), Wm1)
                iyc = jnp.minimum(jnp.maximum(iyi, 0), Hm1)
                row = Si + iyc * Wi + ixc
                g = (nm0 + m) + row * _M
                wt = ((fx if dx else gx0) * (fy if dy else gy0)
                      * aw * jnp.where(ok, 1.0, 0.0))
                idxb[pl.ds(t2 * 16, 16)] = g
                ws.append(wt)
            cp = pltpu.async_copy(table_hbm.at[idxb], rowsb, semb)
            del cp
            return tuple(ws)

        def drain(jj, ws, idxb, rowsb, semb):
            pltpu.make_async_copy(table_hbm.at[idxb], rowsb, semb).wait()
            z = jnp.zeros((16,), jnp.float32)
            a0, a1 = z, z
            for t2 in range(4):
                wvec = ws[t2]
                for sl in range(16):
                    t = t2 * 16 + sl
                    wsc = wvec[sl]
                    a0 = a0 + wsc * rowsb[t, pl.ds(0, 16)]
                    a1 = a1 + wsc * rowsb[t, pl.ds(16, 16)]
            outv[jj, pl.ds(0, 16)] = a0
            outv[jj, pl.ds(16, 16)] = a1

        w_first = compute_issue(0, idx0, rows0, sem0)

        def body(jp, w0):
            j0 = 2 * jp
            w1 = compute_issue(j0 + 1, idx1, rows1, sem1)
            drain(j0, w0, idx0, rows0, sem0)
            w2 = compute_issue(j0 + 2, idx0, rows0, sem0)
            drain(j0 + 1, w1, idx1, rows1, sem1)
            return w2

        w_last = lax.fori_loop(0, JPW // 2 - 1, body, w_first)
        jl = JPW - 2
        w1 = compute_issue(jl + 1, idx1, rows1, sem1)
        drain(jl, w_last, idx0, rows0, sem0)
        drain(jl + 1, w1, idx1, rows1, sem1)
        pltpu.sync_copy(outv, out_hbm.at[pl.ds(wid * JPW, JPW)])

    return k(off, probs, refx, ci, table)


def kernel(query, reference_points, input_flatten, input_spatial_shapes,
           input_level_start_index, q_lidar_indices, W_value, b_value,
           W_off, b_off, W_attn, b_attn, W_out, b_out):
    N, Lq, _ = query.shape
    LEN = input_flatten.shape[1]
    NJ = N * Lq * _M

    value = _mm_bias(input_flatten.reshape(N * LEN, _DM), W_value.T, b_value,
                     (N * LEN) // 17)
    table = value.reshape(N * LEN * _M, _Dh)

    Wofp = (W_off.reshape(_M, _L, _P, 2, _DM)
            .transpose(0, 3, 1, 2, 4).reshape(_M * _LP * 2, _DM))
    bofp = b_off.reshape(_M, _L, _P, 2).transpose(0, 3, 1, 2).reshape(-1)
    Wcat = jnp.concatenate([Wofp, W_attn], axis=0).T
    bcat = jnp.concatenate([bofp, b_attn], axis=0)
    E = np.zeros((128, 2 * _LP), np.float32)
    for l in range(_L):
        for xy in range(2):
            for p in range(_P):
                E[l * 2 + xy, xy * _LP + l * _P + p] = 1.0
    Smat = np.kron(np.eye(_M, dtype=np.float32), np.ones((_LP, _LP), np.float32))
    rp8 = reference_points.reshape(N * Lq, 2 * _L)
    rp_pad = jnp.concatenate(
        [rp8, jnp.zeros((N * Lq, 128 - 2 * _L), jnp.float32)], axis=1)
    off, probs, refx = _qprep(query.reshape(N * Lq, _DM), rp_pad,
                              Wcat, bcat, jnp.asarray(E), jnp.asarray(Smat))
    ss = input_spatial_shapes.astype(jnp.int32)
    lsi = input_level_start_index.astype(jnp.int32)
    ci = jnp.stack([
        jnp.repeat(ss[:, 1], _P),
        jnp.repeat(ss[:, 0], _P),
        jnp.repeat(lsi, _P),
    ])

    attn_out = _sc_attn(off, probs, refx, ci, table, NJ, LEN, Lq)
    out_i = attn_out.reshape(N * Lq, _M * _Dh)
    return _mm_bias(out_i, W_out.T, b_out, Lq).reshape(N, Lq, _DM)


# speedup vs baseline: 77.2749x; 1.5469x over previous
import functools

import jax
import jax.numpy as jnp
import numpy as np
from jax import lax
from jax.experimental import pallas as pl
from jax.experimental.pallas import tpu as pltpu
from jax.experimental.pallas import tpu_sc as plsc

_DM = 256
_M = 8
_L = 4
_P = 4
_Dh = 32
_LP = _L * _P


def _mm_bias(x, w, b, blk_rows):
    R, K = x.shape
    C = w.shape[1]

    def body(x_ref, w_ref, b_ref, o_ref):
        o_ref[...] = (
            jnp.dot(x_ref[...], w_ref[...], preferred_element_type=jnp.float32)
            + b_ref[...]
        )

    return pl.pallas_call(
        body,
        grid=(R // blk_rows,),
        in_specs=[
            pl.BlockSpec((blk_rows, K), lambda i: (i, 0)),
            pl.BlockSpec((K, C), lambda i: (0, 0)),
            pl.BlockSpec((1, C), lambda i: (0, 0)),
        ],
        out_specs=pl.BlockSpec((blk_rows, C), lambda i: (i, 0)),
        out_shape=jax.ShapeDtypeStruct((R, C), jnp.float32),
    )(x, w, b.reshape(1, C))


def _qprep(q2, rp8, Wcat, bcat, E, Smat):
    R = q2.shape[0]
    C = Wcat.shape[1]
    BO = C - _M * _LP

    def body(q_ref, rp_ref, w_ref, b_ref, e_ref, s_ref, o_ref, p_ref, r_ref):
        t = (jnp.dot(q_ref[...], w_ref[...],
                     preferred_element_type=jnp.float32) + b_ref[...])
        lg = t[:, BO:]
        ex = jnp.exp(lg - jnp.max(lg, axis=-1, keepdims=True))
        den = jnp.dot(ex, s_ref[...], preferred_element_type=jnp.float32,
                      precision=jax.lax.Precision.HIGHEST)
        o_ref[...] = t[:, :BO]
        p_ref[...] = ex / den
        r_ref[...] = jnp.dot(rp_ref[...], e_ref[...],
                             preferred_element_type=jnp.float32,
                             precision=jax.lax.Precision.HIGHEST)

    blk = 1024
    return pl.pallas_call(
        body,
        grid=(R // blk,),
        in_specs=[
            pl.BlockSpec((blk, q2.shape[1]), lambda i: (i, 0)),
            pl.BlockSpec((blk, 128), lambda i: (i, 0)),
            pl.BlockSpec(Wcat.shape, lambda i: (0, 0)),
            pl.BlockSpec((1, C), lambda i: (0, 0)),
            pl.BlockSpec(E.shape, lambda i: (0, 0)),
            pl.BlockSpec(Smat.shape, lambda i: (0, 0)),
        ],
        out_specs=[
            pl.BlockSpec((blk, BO), lambda i: (i, 0)),
            pl.BlockSpec((blk, C - BO), lambda i: (i, 0)),
            pl.BlockSpec((blk, 2 * _LP), lambda i: (i, 0)),
        ],
        out_shape=[
            jax.ShapeDtypeStruct((R, BO), jnp.float32),
            jax.ShapeDtypeStruct((R, C - BO), jnp.float32),
            jax.ShapeDtypeStruct((R, 2 * _LP), jnp.float32),
        ],
    )(q2, rp8, Wcat, bcat.reshape(1, C), E, Smat)


def _sc_attn(off, probs, refx, ci, table, NJ, LEN, Lq):
    NW = 32
    JPW = NJ // NW
    QPW = JPW // _M
    mesh = plsc.VectorSubcoreMesh(core_axis_name="c", subcore_axis_name="s",
                                  num_cores=2, num_subcores=16)

    @functools.partial(
        pl.kernel,
        out_type=jax.ShapeDtypeStruct((NJ, _Dh), jnp.float32),
        mesh=mesh,
        compiler_params=pltpu.CompilerParams(use_tc_tiling_on_sc=False),
        scratch_types=[
            pltpu.VMEM((QPW, 256), jnp.float32),
            pltpu.VMEM((QPW, 128), jnp.float32),
            pltpu.VMEM((QPW, 32), jnp.float32),
            pltpu.VMEM((3, 16), jnp.int32),
            pltpu.VMEM((64,), jnp.int32),
            pltpu.VMEM((64,), jnp.int32),
            pltpu.VMEM((64,), jnp.int32),
            pltpu.VMEM((64,), jnp.int32),
            pltpu.VMEM((64, _Dh), jnp.float32),
            pltpu.VMEM((64, _Dh), jnp.float32),
            pltpu.VMEM((64, _Dh), jnp.float32),
            pltpu.VMEM((64, _Dh), jnp.float32),
            pltpu.VMEM((JPW, _Dh), jnp.float32),
            pltpu.SemaphoreType.DMA,
            pltpu.SemaphoreType.DMA,
            pltpu.SemaphoreType.DMA,
            pltpu.SemaphoreType.DMA,
        ],
    )
    def k(off_hbm, aw_hbm, ref_hbm, ci_hbm, table_hbm, out_hbm,
          offv, awv, refv, civ, idx0, idx1, idx2, idx3,
          rows0, rows1, rows2, rows3, outv, sem0, sem1, sem2, sem3):
        c = lax.axis_index("c")
        s = lax.axis_index("s")
        wid = s * 2 + c
        qrow = wid * QPW
        n = qrow // Lq
        pltpu.sync_copy(off_hbm.at[pl.ds(qrow, QPW)], offv)
        pltpu.sync_copy(aw_hbm.at[pl.ds(qrow, QPW)], awv)
        pltpu.sync_copy(ref_hbm.at[pl.ds(qrow, QPW)], refv)
        pltpu.sync_copy(ci_hbm, civ)
        Wi = civ[0, :]
        Hi = civ[1, :]
        Si = civ[2, :]
        Wf = Wi.astype(jnp.float32)
        Hf = Hi.astype(jnp.float32)
        Wm1 = Wi - 1
        Hm1 = Hi - 1
        nm0 = n * (LEN * _M)

        def compute_issue(jj, idxb, rowsb, semb):
            qo = jj // _M
            m = jj - qo * _M
            offx = offv[qo, pl.ds(m * 32, 16)]
            offy = offv[qo, pl.ds(m * 32 + 16, 16)]
            aw = awv[qo, pl.ds(m * 16, 16)]
            rx = refv[qo, pl.ds(0, 16)]
            ry = refv[qo, pl.ds(16, 16)]
            x = rx * Wf + offx - 0.5
            y = ry * Hf + offy - 0.5
            xi = (x + 512.0).astype(jnp.int32)
            yi = (y + 512.0).astype(jnp.int32)
            x0i = xi - 512
            y0i = yi - 512
            fx = x - x0i.astype(jnp.float32)
            fy = y - y0i.astype(jnp.float32)
            gx0 = 1.0 - fx
            gy0 = 1.0 - fy
            ws = []
            for t2, (dy, dx) in enumerate(((0, 0), (0, 1), (1, 0), (1, 1))):
                ixi = x0i + dx
                iyi = y0i + dy
                ok = (ixi >= 0) & (ixi <= Wm1) & (iyi >= 0) & (iyi <= Hm1)
                ixc = jnp.minimum(jnp.maximum(ixi, 0), Wm1)
                iyc = jnp.minimum(jnp.maximum(iyi, 0), Hm1)
                row = Si + iyc * Wi + ixc
                g = (nm0 + m) + row * _M
                wt = ((fx if dx else gx0) * (fy if dy else gy0)
                      * aw * jnp.where(ok, 1.0, 0.0))
                idxb[pl.ds(t2 * 16, 16)] = g
                ws.append(wt)
            cp = pltpu.async_copy(table_hbm.at[idxb], rowsb, semb)
            del cp
            return tuple(ws)

        def drain(jj, ws, idxb, rowsb, semb):
            pltpu.make_async_copy(table_hbm.at[idxb], rowsb, semb).wait()
            z = jnp.zeros((16,), jnp.float32)
            a0, a1 = z, z
            for t2 in range(4):
                wvec = ws[t2]
                for sl in range(16):
                    t = t2 * 16 + sl
                    wsc = wvec[sl]
                    a0 = a0 + wsc * rowsb[t, pl.ds(0, 16)]
                    a1 = a1 + wsc * rowsb[t, pl.ds(16, 16)]
            outv[jj, pl.ds(0, 16)] = a0
            outv[jj, pl.ds(16, 16)] = a1

        bufs = ((idx0, rows0, sem0), (idx1, rows1, sem1),
                (idx2, rows2, sem2), (idx3, rows3, sem3))
        init = (compute_issue(0, *bufs[0]),
                compute_issue(1, *bufs[1]),
                compute_issue(2, *bufs[2]))

        def body(jp, pending):
            pa, pb, pc = pending
            base = 4 * jp
            w3 = compute_issue(base + 3, *bufs[3])
            drain(base + 0, pa, *bufs[0])
            w4 = compute_issue(base + 4, *bufs[0])
            drain(base + 1, pb, *bufs[1])
            w5 = compute_issue(base + 5, *bufs[1])
            drain(base + 2, pc, *bufs[2])
            w6 = compute_issue(base + 6, *bufs[2])
            drain(base + 3, w3, *bufs[3])
            return (w4, w5, w6)

        pa, pb, pc = lax.fori_loop(0, JPW // 4 - 1, body, init)
        w3 = compute_issue(JPW - 1, *bufs[3])
        drain(JPW - 4, pa, *bufs[0])
        drain(JPW - 3, pb, *bufs[1])
        drain(JPW - 2, pc, *bufs[2])
        drain(JPW - 1, w3, *bufs[3])
        pltpu.sync_copy(outv, out_hbm.at[pl.ds(wid * JPW, JPW)])

    return k(off, probs, refx, ci, table)


def kernel(query, reference_points, input_flatten, input_spatial_shapes,
           input_level_start_index, q_lidar_indices, W_value, b_value,
           W_off, b_off, W_attn, b_attn, W_out, b_out):
    N, Lq, _ = query.shape
    LEN = input_flatten.shape[1]
    NJ = N * Lq * _M

    value = _mm_bias(input_flatten.reshape(N * LEN, _DM), W_value.T, b_value,
                     (N * LEN) // 17)
    table = value.reshape(N * LEN * _M, _Dh)

    Wofp = (W_off.reshape(_M, _L, _P, 2, _DM)
            .transpose(0, 3, 1, 2, 4).reshape(_M * _LP * 2, _DM))
    bofp = b_off.reshape(_M, _L, _P, 2).transpose(0, 3, 1, 2).reshape(-1)
    Wcat = jnp.concatenate([Wofp, W_attn], axis=0).T
    bcat = jnp.concatenate([bofp, b_attn], axis=0)
    E = np.zeros((128, 2 * _LP), np.float32)
    for l in range(_L):
        for xy in range(2):
            for p in range(_P):
                E[l * 2 + xy, xy * _LP + l * _P + p] = 1.0
    Smat = np.kron(np.eye(_M, dtype=np.float32), np.ones((_LP, _LP), np.float32))
    rp8 = reference_points.reshape(N * Lq, 2 * _L)
    rp_pad = jnp.concatenate(
        [rp8, jnp.zeros((N * Lq, 128 - 2 * _L), jnp.float32)], axis=1)
    off, probs, refx = _qprep(query.reshape(N * Lq, _DM), rp_pad,
                              Wcat, bcat, jnp.asarray(E), jnp.asarray(Smat))
    ss = input_spatial_shapes.astype(jnp.int32)
    lsi = input_level_start_index.astype(jnp.int32)
    ci = jnp.stack([
        jnp.repeat(ss[:, 1], _P),
        jnp.repeat(ss[:, 0], _P),
        jnp.repeat(lsi, _P),
    ])

    attn_out = _sc_attn(off, probs, refx, ci, table, NJ, LEN, Lq)
    out_i = attn_out.reshape(N * Lq, _M * _Dh)
    return _mm_bias(out_i, W_out.T, b_out, Lq).reshape(N, Lq, _DM)
